# baseline (speedup 1.0000x reference)
import functools

import jax
import jax.numpy as jnp
from jax import lax
from jax.experimental import pallas as pl
from jax.experimental.pallas import tpu as pltpu
from jax.experimental.pallas import tpu_sc as plsc

B = 4096
L = 200
F = 4
NC, NS = 2, 16
NW = NC * NS
CW = B // NW
LCH = 40
NCHUNK = L // LCH
NV = CW // 16
WIDTHS = (12, 31, 7, 24)

_mesh = plsc.VectorSubcoreMesh(core_axis_name="c", subcore_axis_name="s")


@functools.partial(
    pl.kernel,
    out_type=jax.ShapeDtypeStruct((L, B), jnp.float32),
    mesh=_mesh,
    compiler_params=pltpu.CompilerParams(needs_layout_passes=False),
    scratch_types=[
        pltpu.VMEM((LCH, F * CW), jnp.int32),
        pltpu.VMEM((LCH, F * CW), jnp.int32),
        pltpu.VMEM((L, CW), jnp.float32),
        pltpu.SemaphoreType.DMA,
        pltpu.SemaphoreType.DMA,
        pltpu.SemaphoreType.DMA,
    ],
)
def _t2v_sc(xs_hbm, out_hbm, in0, in1, cnt_v, si0, si1, so):
    wid = lax.axis_index("s") * NC + lax.axis_index("c")
    col = wid * CW
    inbufs, isems = (in0, in1), (si0, si1)

    one = jnp.full((16,), 1.0, jnp.float32)
    zero = jnp.full((16,), 0.0, jnp.float32)

    def start_in(c):
        return pltpu.async_copy(
            xs_hbm.at[pl.ds(c * LCH, LCH), wid],
            inbufs[c % 2], isems[c % 2])

    def process(c, acc):
        ib = inbufs[c % 2]

        def row_body(r, acc):
            out_acc = []
            for v in range(NV):
                cnt = zero
                for f, w in enumerate(WIDTHS):
                    vu = plsc.bitcast(
                        ib[r, pl.ds(f * CW + v * 16, 16)], jnp.uint32)
                    cnt = cnt + jnp.where(vu < jnp.uint32(w), one, zero)
                cnt_v[c * LCH + r, pl.ds(v * 16, 16)] = cnt
                out_acc.append(acc[v] + cnt * cnt)
            return tuple(out_acc)

        return lax.fori_loop(0, LCH, row_body, acc)

    cps = [None] * NCHUNK
    cps[0] = start_in(0)
    acc = (zero,) * NV
    for c in range(NCHUNK):
        if c + 1 < NCHUNK:
            cps[c + 1] = start_in(c + 1)
        cps[c].wait()
        acc = process(c, acc)

    gs = []
    for v in range(NV):
        t = acc[v]
        gi = jnp.int32(0x5F3759DF) - (plsc.bitcast(t, jnp.int32) >> 1)
        g = plsc.bitcast(gi, jnp.float32)
        for _ in range(3):
            g = g * (1.5 - 0.5 * t * g * g)
        gs.append(g)

    def scale_body(r, carry):
        for v in range(NV):
            sl = pl.ds(v * 16, 16)
            cnt_v[r, sl] = cnt_v[r, sl] * gs[v]
        return carry

    lax.fori_loop(0, L, scale_body, 0)
    pltpu.async_copy(cnt_v, out_hbm.at[:, pl.ds(col, CW)], so).wait()


def kernel(x):
    xs = (x.astype(jnp.int32).reshape(NW, CW, L, F)
          .transpose(2, 0, 3, 1).reshape(L, NW, F * CW))
    return _t2v_sc(xs).T


# speedup vs baseline: 1.3429x; 1.3429x over previous
import functools

import jax
import jax.numpy as jnp
from jax import lax
from jax.experimental import pallas as pl
from jax.experimental.pallas import tpu as pltpu
from jax.experimental.pallas import tpu_sc as plsc

B = 4096
L = 200
F = 4
NC, NS = 2, 16
NW = NC * NS
CW = B // NW
LCH = 40
NCHUNK = L // LCH
NV = CW // 16
WIDTHS = (12, 31, 7, 24)

_mesh = plsc.VectorSubcoreMesh(core_axis_name="c", subcore_axis_name="s")


@functools.partial(
    pl.kernel,
    out_type=jax.ShapeDtypeStruct((L, B), jnp.float32),
    mesh=_mesh,
    compiler_params=pltpu.CompilerParams(needs_layout_passes=False),
    scratch_types=[
        pltpu.VMEM((LCH, F, CW), jnp.int32),
        pltpu.VMEM((LCH, F, CW), jnp.int32),
        pltpu.VMEM((L, CW), jnp.float32),
        pltpu.SemaphoreType.DMA,
        pltpu.SemaphoreType.DMA,
        pltpu.SemaphoreType.DMA,
    ],
)
def _t2v_sc(xs_hbm, out_hbm, in0, in1, cnt_v, si0, si1, so):
    wid = lax.axis_index("s") * NC + lax.axis_index("c")
    col = wid * CW
    inbufs, isems = (in0, in1), (si0, si1)

    one = jnp.full((16,), 1.0, jnp.float32)
    zero = jnp.full((16,), 0.0, jnp.float32)

    def start_in(c):
        return pltpu.async_copy(
            xs_hbm.at[pl.ds(c * LCH, LCH), wid],
            inbufs[c % 2], isems[c % 2])

    def process(c, acc):
        ib = inbufs[c % 2]

        def row_body(r, acc):
            out_acc = []
            for v in range(NV):
                cnt = zero
                for f, w in enumerate(WIDTHS):
                    vu = plsc.bitcast(
                        ib[r, f, pl.ds(v * 16, 16)], jnp.uint32)
                    cnt = cnt + jnp.where(vu < jnp.uint32(w), one, zero)
                cnt_v[c * LCH + r, pl.ds(v * 16, 16)] = cnt
                out_acc.append(acc[v] + cnt * cnt)
            return tuple(out_acc)

        return lax.fori_loop(0, LCH, row_body, acc)

    cps = [None] * NCHUNK
    cps[0] = start_in(0)
    acc = (zero,) * NV
    for c in range(NCHUNK):
        if c + 1 < NCHUNK:
            cps[c + 1] = start_in(c + 1)
        cps[c].wait()
        acc = process(c, acc)

    gs = []
    for v in range(NV):
        t = acc[v]
        gi = jnp.int32(0x5F3759DF) - (plsc.bitcast(t, jnp.int32) >> 1)
        g = plsc.bitcast(gi, jnp.float32)
        for _ in range(3):
            g = g * (1.5 - 0.5 * t * g * g)
        gs.append(g)

    def scale_body(r, carry):
        for v in range(NV):
            sl = pl.ds(v * 16, 16)
            cnt_v[r, sl] = cnt_v[r, sl] * gs[v]
        return carry

    lax.fori_loop(0, L, scale_body, 0)
    pltpu.async_copy(cnt_v, out_hbm.at[:, pl.ds(col, CW)], so).wait()


def kernel(x):
    xs = (x.astype(jnp.int32).reshape(NW, CW, L, F)
          .transpose(2, 0, 3, 1))
    return _t2v_sc(xs).T


# speedup vs baseline: 1.3619x; 1.0141x over previous
import functools

import jax
import jax.numpy as jnp
from jax import lax
from jax.experimental import pallas as pl
from jax.experimental.pallas import tpu as pltpu
from jax.experimental.pallas import tpu_sc as plsc

B = 4096
L = 200
F = 4
NC, NS = 2, 16
NW = NC * NS
CW = B // NW
LCH = 40
NCHUNK = L // LCH
NV = CW // 16
WIDTHS = (12, 31, 7, 24)

_mesh = plsc.VectorSubcoreMesh(core_axis_name="c", subcore_axis_name="s")


@functools.partial(
    pl.kernel,
    out_type=jax.ShapeDtypeStruct((L, B), jnp.float32),
    mesh=_mesh,
    compiler_params=pltpu.CompilerParams(needs_layout_passes=False),
    scratch_types=[
        pltpu.VMEM((LCH, F, CW), jnp.int32),
        pltpu.VMEM((LCH, F, CW), jnp.int32),
        pltpu.VMEM((L, CW), jnp.float32),
        pltpu.SemaphoreType.DMA,
        pltpu.SemaphoreType.DMA,
        pltpu.SemaphoreType.DMA,
        pltpu.SemaphoreType.DMA,
    ],
)
def _t2v_sc(xs_hbm, out_hbm, in0, in1, cnt_v, si0, si1, so0, so1):
    wid = lax.axis_index("s") * NC + lax.axis_index("c")
    col = wid * CW
    inbufs, isems = (in0, in1), (si0, si1)

    one = jnp.full((16,), 1.0, jnp.float32)
    zero = jnp.full((16,), 0.0, jnp.float32)

    def start_in(c):
        return pltpu.async_copy(
            xs_hbm.at[pl.ds(c * LCH, LCH), wid],
            inbufs[c % 2], isems[c % 2])

    def process(c, acc):
        ib = inbufs[c % 2]

        def row_body(r, acc):
            out_acc = []
            for v in range(NV):
                ms = []
                for f, w in enumerate(WIDTHS):
                    vu = plsc.bitcast(
                        ib[r, f, pl.ds(v * 16, 16)], jnp.uint32)
                    ms.append(jnp.where(vu < jnp.uint32(w), one, zero))
                cnt = (ms[0] + ms[1]) + (ms[2] + ms[3])
                cnt_v[c * LCH + r, pl.ds(v * 16, 16)] = cnt
                out_acc.append(acc[v] + cnt * cnt)
            return tuple(out_acc)

        return lax.fori_loop(0, LCH, row_body, acc)

    cps = [None] * NCHUNK
    cps[0] = start_in(0)
    acc = (zero,) * NV
    for c in range(NCHUNK):
        if c + 1 < NCHUNK:
            cps[c + 1] = start_in(c + 1)
        cps[c].wait()
        acc = process(c, acc)

    gs = []
    for v in range(NV):
        t = acc[v]
        gi = jnp.int32(0x5F3759DF) - (plsc.bitcast(t, jnp.int32) >> 1)
        g = plsc.bitcast(gi, jnp.float32)
        for _ in range(3):
            g = g * (1.5 - 0.5 * t * g * g)
        gs.append(g)

    def scale_body(r, carry):
        for v in range(NV):
            sl = pl.ds(v * 16, 16)
            cnt_v[r, sl] = cnt_v[r, sl] * gs[v]
        return carry

    osems = (so0, so1)
    ops = [None] * NCHUNK
    for c in range(NCHUNK):
        lax.fori_loop(c * LCH, (c + 1) * LCH, scale_body, 0)
        if c >= 2:
            ops[c - 2].wait()
        ops[c] = pltpu.async_copy(
            cnt_v.at[pl.ds(c * LCH, LCH), :],
            out_hbm.at[pl.ds(c * LCH, LCH), pl.ds(col, CW)],
            osems[c % 2])
    ops[NCHUNK - 2].wait()
    ops[NCHUNK - 1].wait()


def kernel(x):
    xs = (x.astype(jnp.int32).reshape(NW, CW, L, F)
          .transpose(2, 0, 3, 1))
    return _t2v_sc(xs).T
